# baseline (speedup 1.0000x reference)
import functools

import jax
import jax.numpy as jnp
from jax import lax
from jax.experimental import pallas as pl
from jax.experimental.pallas import tpu as pltpu
from jax.experimental.pallas import tpu_sc as plsc

N = 10000
E = 320000
NC = 2
NS = 16
L = 16
NW = NC * NS
NP = 10240
BB = 128
EPT = 10240
EPTL = E - (NW - 1) * EPT
NB = EPT // BB
NBL = EPTL // BB
G = 20
NSL = NP // NS
MROWS = 1024


def _sc_mesh():
    return plsc.VectorSubcoreMesh(core_axis_name="c", subcore_axis_name="s")


_SC_PARAMS = pltpu.CompilerParams(needs_layout_passes=False)


def _zero_acc_slice(zero_v, acc_sh, sid):
    for i in range(NSL // L):
        zero_v[pl.ds(i * L, L)] = jnp.zeros((L,), jnp.float32)
    pltpu.sync_copy(zero_v, acc_sh.at[pl.ds(sid * NSL, NSL)])


def _stage_edges(ei_hbm, row, buf_v, wid):
    last = wid == NW - 1

    @pl.when(jnp.logical_not(last))
    def _():
        pltpu.sync_copy(ei_hbm.at[row, pl.ds(wid * EPT, EPT)], buf_v)

    @pl.when(last)
    def _():
        pltpu.sync_copy(ei_hbm.at[row, pl.ds(E - EPTL, EPTL)],
                        buf_v.at[pl.ds(0, EPTL)])

    return jnp.where(last, NBL // G, NB // G)


def _fast_rsqrt(d):
    i = plsc.bitcast(d, jnp.int32)
    i = jnp.int32(0x5F3759DF) - lax.shift_right_arithmetic(i, 1)
    y = plsc.bitcast(i, jnp.float32)
    y = y * (1.5 - 0.5 * d * y * y)
    y = y * (1.5 - 0.5 * d * y * y)
    y = y * (1.5 - 0.5 * d * y * y)
    return y


def _deg_body(ei_hbm, out_hbm, didx_v, ones_v, zero_v, acc_sh, sem):
    cid = lax.axis_index("c")
    sid = lax.axis_index("s")
    wid = cid * NS + sid
    ngroups = _stage_edges(ei_hbm, 1, didx_v, wid)
    for i in range(BB // L):
        ones_v[pl.ds(i * L, L)] = jnp.ones((L,), jnp.float32)
    _zero_acc_slice(zero_v, acc_sh, sid)
    plsc.subcore_barrier()

    def group(g, carry):
        cps = [
            pltpu.async_copy(ones_v,
                             acc_sh.at[didx_v.at[pl.ds((g * G + jj) * BB, BB)]],
                             sem, add=True)
            for jj in range(G)
        ]
        for cp in cps:
            cp.wait()
        return carry

    lax.fori_loop(0, ngroups, group, 0)
    plsc.subcore_barrier()
    pltpu.sync_copy(acc_sh.at[pl.ds(sid * NSL, NSL)],
                    out_hbm.at[cid, pl.ds(sid * NSL, NSL)])


def _degree_partials(edge_index):
    return pl.kernel(
        _deg_body,
        out_type=jax.ShapeDtypeStruct((NC, NP), jnp.float32),
        mesh=_sc_mesh(),
        compiler_params=_SC_PARAMS,
        scratch_types=[
            pltpu.VMEM((EPT,), jnp.int32),
            pltpu.VMEM((BB,), jnp.float32),
            pltpu.VMEM((NSL,), jnp.float32),
            pltpu.VMEM_SHARED((NP,), jnp.float32),
            pltpu.SemaphoreType.DMA,
        ],
    )(edge_index)


def _msg_body(ei_hbm, traw_hbm, degp_hbm, out_hbm,
              sidx_v, didx_v, vals_v, t_v, dg_v, zero_v, acc_sh, sem):
    cid = lax.axis_index("c")
    sid = lax.axis_index("s")
    wid = cid * NS + sid
    tcp = pltpu.async_copy(traw_hbm, t_v, sem)
    dcp = pltpu.async_copy(degp_hbm, dg_v, sem)
    ngroups = _stage_edges(ei_hbm, 0, sidx_v, wid)
    _stage_edges(ei_hbm, 1, didx_v, wid)
    _zero_acc_slice(zero_v, acc_sh, sid)
    tcp.wait()
    dcp.wait()
    _BU = 8

    def build(i, carry):
        for q in range(_BU):
            o = (i * _BU + q) * L
            d = dg_v[0, pl.ds(o, L)] + dg_v[1, pl.ds(o, L)] + 1.0
            t_v[pl.ds(o, L)] = t_v[pl.ds(o, L)] * _fast_rsqrt(d)
        return carry

    lax.fori_loop(0, NP // L // _BU, build, 0)
    plsc.subcore_barrier()

    def group(g, carry):
        cps = []
        for jj in range(G):
            j = g * G + jj
            for k in range(BB // L):
                off = j * BB + k * L
                si = sidx_v[pl.ds(off, L)]
                vals_v[pl.ds(off, L)] = plsc.load_gather(t_v, [si])
            cps.append(
                pltpu.async_copy(vals_v.at[pl.ds(j * BB, BB)],
                                 acc_sh.at[didx_v.at[pl.ds(j * BB, BB)]],
                                 sem, add=True))
        for cp in cps:
            cp.wait()
        return carry

    lax.fori_loop(0, ngroups, group, 0)
    plsc.subcore_barrier()
    pltpu.sync_copy(acc_sh.at[pl.ds(sid * NSL, NSL)],
                    out_hbm.at[cid, pl.ds(sid * NSL, NSL)])


def _message_partials(edge_index, t_raw, degp):
    return pl.kernel(
        _msg_body,
        out_type=jax.ShapeDtypeStruct((NC, NP), jnp.float32),
        mesh=_sc_mesh(),
        compiler_params=_SC_PARAMS,
        scratch_types=[
            pltpu.VMEM((EPT,), jnp.int32),
            pltpu.VMEM((EPT,), jnp.int32),
            pltpu.VMEM((EPT,), jnp.float32),
            pltpu.VMEM((NP,), jnp.float32),
            pltpu.VMEM((NC, NP), jnp.float32),
            pltpu.VMEM((NSL,), jnp.float32),
            pltpu.VMEM_SHARED((NP,), jnp.float32),
            pltpu.SemaphoreType.DMA,
        ],
    )(edge_index, t_raw, degp)


def _mid_body(x_ref, wp_ref, bp_ref, wg_ref, wo_ref, traw_ref):
    u = jnp.dot(wo_ref[...][0, :], wg_ref[...])
    h = lax.dot_general(x_ref[...], wp_ref[...],
                        (((1,), (1,)), ((), ())),
                        preferred_element_type=jnp.float32)
    h = jnp.maximum(h + bp_ref[...][None, :], 0.0)
    traw_ref[...] = jnp.sum(h * u[None, :], axis=1)


def _tc_mid(x, W_proj, b_proj, W_gcn, W_out):
    return pl.pallas_call(
        _mid_body,
        grid=(NP // MROWS,),
        in_specs=[
            pl.BlockSpec((MROWS, 128), lambda i: (i, 0)),
            pl.BlockSpec((64, 128), lambda i: (0, 0)),
            pl.BlockSpec((64,), lambda i: (0,)),
            pl.BlockSpec((32, 64), lambda i: (0, 0)),
            pl.BlockSpec((1, 32), lambda i: (0, 0)),
        ],
        out_specs=pl.BlockSpec((MROWS,), lambda i: (i,)),
        out_shape=jax.ShapeDtypeStruct((NP,), jnp.float32),
    )(x, W_proj, b_proj, W_gcn, W_out)


def _final_body(traw_ref, degp_ref, sp_ref, wo_ref, bg_ref, bo_ref, out_ref):
    c = jnp.sum(wo_ref[...][0, :] * bg_ref[...]) + jnp.sum(bo_ref[...])
    deg = degp_ref[0, :] + degp_ref[1, :] + 1.0
    dinv = lax.rsqrt(deg)
    t = dinv * traw_ref[...]
    out_ref[...] = dinv * (sp_ref[0, :] + sp_ref[1, :] + t) + c


def _tc_final(t_raw, degp, sp, W_out, b_gcn, b_out):
    return pl.pallas_call(
        _final_body,
        out_shape=jax.ShapeDtypeStruct((NP,), jnp.float32),
    )(t_raw, degp, sp, W_out, b_gcn, b_out)


@jax.jit
def kernel(x, edge_index, W_proj, b_proj, W_gcn, b_gcn, W_out, b_out):
    degp = _degree_partials(edge_index)
    t_raw = _tc_mid(x, W_proj, b_proj, W_gcn, W_out)
    sp = _message_partials(edge_index, t_raw, degp)
    out_full = _tc_final(t_raw, degp, sp, W_out, b_gcn, b_out)
    return out_full[:N, None]


# speedup vs baseline: 1.2614x; 1.2614x over previous
import functools

import jax
import jax.numpy as jnp
from jax import lax
from jax.experimental import pallas as pl
from jax.experimental.pallas import tpu as pltpu
from jax.experimental.pallas import tpu_sc as plsc

N = 10000
E = 320000
NC = 2
NS = 16
L = 16
NW = NC * NS
NP = 10240
BB = 128
EPT = 10240
EPTL = E - (NW - 1) * EPT
NB = EPT // BB
NBL = EPTL // BB
G = 20
NSL = NP // NS
MROWS = 1024


def _sc_mesh():
    return plsc.VectorSubcoreMesh(core_axis_name="c", subcore_axis_name="s")


_SC_PARAMS = pltpu.CompilerParams(needs_layout_passes=False)


def _zero_acc_slice(zero_v, acc_sh, sid):
    for i in range(NSL // L):
        zero_v[pl.ds(i * L, L)] = jnp.zeros((L,), jnp.float32)
    pltpu.sync_copy(zero_v, acc_sh.at[pl.ds(sid * NSL, NSL)])


def _stage_edges(ei_hbm, row, buf_v, wid):
    last = wid == NW - 1

    @pl.when(jnp.logical_not(last))
    def _():
        pltpu.sync_copy(ei_hbm.at[row, pl.ds(wid * EPT, EPT)], buf_v)

    @pl.when(last)
    def _():
        pltpu.sync_copy(ei_hbm.at[row, pl.ds(E - EPTL, EPTL)],
                        buf_v.at[pl.ds(0, EPTL)])

    return jnp.where(last, NBL // G, NB // G)


def _fast_rsqrt(d):
    i = plsc.bitcast(d, jnp.int32)
    i = jnp.int32(0x5F3759DF) - lax.shift_right_arithmetic(i, 1)
    y = plsc.bitcast(i, jnp.float32)
    y = y * (1.5 - 0.5 * d * y * y)
    y = y * (1.5 - 0.5 * d * y * y)
    y = y * (1.5 - 0.5 * d * y * y)
    return y


def _deg_body(ei_hbm, out_hbm, didx_v, ones_v, zero_v, acc_sh, sem):
    cid = lax.axis_index("c")
    sid = lax.axis_index("s")
    wid = cid * NS + sid
    ngroups = _stage_edges(ei_hbm, 1, didx_v, wid)
    for i in range(BB // L):
        ones_v[pl.ds(i * L, L)] = jnp.ones((L,), jnp.float32)
    _zero_acc_slice(zero_v, acc_sh, sid)
    plsc.subcore_barrier()

    def group(g, carry):
        cps = [
            pltpu.async_copy(ones_v,
                             acc_sh.at[didx_v.at[pl.ds((g * G + jj) * BB, BB)]],
                             sem, add=True)
            for jj in range(G)
        ]
        for cp in cps:
            cp.wait()
        return carry

    lax.fori_loop(0, ngroups, group, 0)
    plsc.subcore_barrier()
    pltpu.sync_copy(acc_sh.at[pl.ds(sid * NSL, NSL)],
                    out_hbm.at[cid, pl.ds(sid * NSL, NSL)])


def _degree_partials(edge_index):
    return pl.kernel(
        _deg_body,
        out_type=jax.ShapeDtypeStruct((NC, NP), jnp.float32),
        mesh=_sc_mesh(),
        compiler_params=_SC_PARAMS,
        scratch_types=[
            pltpu.VMEM((EPT,), jnp.int32),
            pltpu.VMEM((BB,), jnp.float32),
            pltpu.VMEM((NSL,), jnp.float32),
            pltpu.VMEM_SHARED((NP,), jnp.float32),
            pltpu.SemaphoreType.DMA,
        ],
    )(edge_index)


def _msg_body(ei_hbm, traw_hbm, degp_hbm, out_hbm,
              sidx_v, didx_v, vals_v, t_v, dg_v, zero_v, acc_sh, sem):
    cid = lax.axis_index("c")
    sid = lax.axis_index("s")
    wid = cid * NS + sid
    tcp = pltpu.async_copy(traw_hbm, t_v, sem)
    dcp = pltpu.async_copy(degp_hbm, dg_v, sem)
    ngroups = _stage_edges(ei_hbm, 0, sidx_v, wid)
    _stage_edges(ei_hbm, 1, didx_v, wid)
    _zero_acc_slice(zero_v, acc_sh, sid)
    tcp.wait()
    dcp.wait()
    @plsc.parallel_loop(0, NP // L, 1, unroll=8)
    def _build(i):
        o = i * L
        d = dg_v[0, pl.ds(o, L)] + dg_v[1, pl.ds(o, L)] + 1.0
        t_v[pl.ds(o, L)] = t_v[pl.ds(o, L)] * _fast_rsqrt(d)
    plsc.subcore_barrier()

    def group(g, carry):
        cps = []
        for jj in range(G):
            j = g * G + jj
            for k in range(BB // L):
                off = j * BB + k * L
                si = sidx_v[pl.ds(off, L)]
                vals_v[pl.ds(off, L)] = plsc.load_gather(t_v, [si])
            cps.append(
                pltpu.async_copy(vals_v.at[pl.ds(j * BB, BB)],
                                 acc_sh.at[didx_v.at[pl.ds(j * BB, BB)]],
                                 sem, add=True))
        for cp in cps:
            cp.wait()
        return carry

    lax.fori_loop(0, ngroups, group, 0)
    plsc.subcore_barrier()
    pltpu.sync_copy(acc_sh.at[pl.ds(sid * NSL, NSL)],
                    out_hbm.at[cid, pl.ds(sid * NSL, NSL)])


def _message_partials(edge_index, t_raw, degp):
    return pl.kernel(
        _msg_body,
        out_type=jax.ShapeDtypeStruct((NC, NP), jnp.float32),
        mesh=_sc_mesh(),
        compiler_params=_SC_PARAMS,
        scratch_types=[
            pltpu.VMEM((EPT,), jnp.int32),
            pltpu.VMEM((EPT,), jnp.int32),
            pltpu.VMEM((EPT,), jnp.float32),
            pltpu.VMEM((NP,), jnp.float32),
            pltpu.VMEM((NC, NP), jnp.float32),
            pltpu.VMEM((NSL,), jnp.float32),
            pltpu.VMEM_SHARED((NP,), jnp.float32),
            pltpu.SemaphoreType.DMA,
        ],
    )(edge_index, t_raw, degp)


def _mid_body(x_ref, wp_ref, bp_ref, wg_ref, wo_ref, traw_ref):
    u = jnp.dot(wo_ref[...][0, :], wg_ref[...])
    h = lax.dot_general(x_ref[...], wp_ref[...],
                        (((1,), (1,)), ((), ())),
                        preferred_element_type=jnp.float32)
    h = jnp.maximum(h + bp_ref[...][None, :], 0.0)
    traw_ref[...] = jnp.sum(h * u[None, :], axis=1)


def _tc_mid(x, W_proj, b_proj, W_gcn, W_out):
    return pl.pallas_call(
        _mid_body,
        grid=(NP // MROWS,),
        in_specs=[
            pl.BlockSpec((MROWS, 128), lambda i: (i, 0)),
            pl.BlockSpec((64, 128), lambda i: (0, 0)),
            pl.BlockSpec((64,), lambda i: (0,)),
            pl.BlockSpec((32, 64), lambda i: (0, 0)),
            pl.BlockSpec((1, 32), lambda i: (0, 0)),
        ],
        out_specs=pl.BlockSpec((MROWS,), lambda i: (i,)),
        out_shape=jax.ShapeDtypeStruct((NP,), jnp.float32),
    )(x, W_proj, b_proj, W_gcn, W_out)


def _final_body(traw_ref, degp_ref, sp_ref, wo_ref, bg_ref, bo_ref, out_ref):
    c = jnp.sum(wo_ref[...][0, :] * bg_ref[...]) + jnp.sum(bo_ref[...])
    deg = degp_ref[0, :] + degp_ref[1, :] + 1.0
    dinv = lax.rsqrt(deg)
    t = dinv * traw_ref[...]
    out_ref[...] = dinv * (sp_ref[0, :] + sp_ref[1, :] + t) + c


def _tc_final(t_raw, degp, sp, W_out, b_gcn, b_out):
    return pl.pallas_call(
        _final_body,
        out_shape=jax.ShapeDtypeStruct((NP,), jnp.float32),
    )(t_raw, degp, sp, W_out, b_gcn, b_out)


@jax.jit
def kernel(x, edge_index, W_proj, b_proj, W_gcn, b_gcn, W_out, b_out):
    degp = _degree_partials(edge_index)
    t_raw = _tc_mid(x, W_proj, b_proj, W_gcn, W_out)
    sp = _message_partials(edge_index, t_raw, degp)
    out_full = _tc_final(t_raw, degp, sp, W_out, b_gcn, b_out)
    return out_full[:N, None]


# speedup vs baseline: 1.2839x; 1.0178x over previous
import functools

import jax
import jax.numpy as jnp
from jax import lax
from jax.experimental import pallas as pl
from jax.experimental.pallas import tpu as pltpu
from jax.experimental.pallas import tpu_sc as plsc

N = 10000
E = 320000
NC = 2
NS = 16
L = 16
NW = NC * NS
NP = 10240
BB = 128
EPT = 10240
EPTL = E - (NW - 1) * EPT
NB = EPT // BB
NBL = EPTL // BB
G = 20
NSL = NP // NS
MROWS = 1024


def _sc_mesh():
    return plsc.VectorSubcoreMesh(core_axis_name="c", subcore_axis_name="s")


_SC_PARAMS = pltpu.CompilerParams(needs_layout_passes=False)


def _zero_acc_slice(zero_v, acc_sh, sid):
    for i in range(NSL // L):
        zero_v[pl.ds(i * L, L)] = jnp.zeros((L,), jnp.float32)
    pltpu.sync_copy(zero_v, acc_sh.at[pl.ds(sid * NSL, NSL)])


def _stage_edges(ei_hbm, row, buf_v, wid):
    last = wid == NW - 1

    @pl.when(jnp.logical_not(last))
    def _():
        pltpu.sync_copy(ei_hbm.at[row, pl.ds(wid * EPT, EPT)], buf_v)

    @pl.when(last)
    def _():
        pltpu.sync_copy(ei_hbm.at[row, pl.ds(E - EPTL, EPTL)],
                        buf_v.at[pl.ds(0, EPTL)])

    return jnp.where(last, NBL // G, NB // G)


def _fast_rsqrt(d):
    i = plsc.bitcast(d, jnp.int32)
    i = jnp.int32(0x5F3759DF) - lax.shift_right_arithmetic(i, 1)
    y = plsc.bitcast(i, jnp.float32)
    y = y * (1.5 - 0.5 * d * y * y)
    y = y * (1.5 - 0.5 * d * y * y)
    y = y * (1.5 - 0.5 * d * y * y)
    return y


def _deg_body(ei_hbm, out_hbm, didx_v, ones_v, zero_v, acc_sh, sem):
    cid = lax.axis_index("c")
    sid = lax.axis_index("s")
    wid = cid * NS + sid
    ngroups = _stage_edges(ei_hbm, 1, didx_v, wid)
    for i in range(BB // L):
        ones_v[pl.ds(i * L, L)] = jnp.ones((L,), jnp.float32)
    _zero_acc_slice(zero_v, acc_sh, sid)
    plsc.subcore_barrier()

    def group(g, carry):
        cps = [
            pltpu.async_copy(ones_v,
                             acc_sh.at[didx_v.at[pl.ds((g * G + jj) * BB, BB)]],
                             sem, add=True)
            for jj in range(G)
        ]
        for cp in cps:
            cp.wait()
        return carry

    lax.fori_loop(0, ngroups, group, 0)
    plsc.subcore_barrier()
    pltpu.sync_copy(acc_sh.at[pl.ds(sid * NSL, NSL)],
                    out_hbm.at[cid, pl.ds(sid * NSL, NSL)])


def _degree_partials(edge_index):
    return pl.kernel(
        _deg_body,
        out_type=jax.ShapeDtypeStruct((NC, NP), jnp.float32),
        mesh=_sc_mesh(),
        compiler_params=_SC_PARAMS,
        scratch_types=[
            pltpu.VMEM((EPT,), jnp.int32),
            pltpu.VMEM((BB,), jnp.float32),
            pltpu.VMEM((NSL,), jnp.float32),
            pltpu.VMEM_SHARED((NP,), jnp.float32),
            pltpu.SemaphoreType.DMA,
        ],
    )(edge_index)


def _msg_body(ei_hbm, traw_hbm, degp_hbm, out_hbm,
              sidx_v, didx_v, vals_v, t_v, dg_v, zero_v, acc_sh, sem):
    cid = lax.axis_index("c")
    sid = lax.axis_index("s")
    wid = cid * NS + sid
    tcp = pltpu.async_copy(traw_hbm, t_v, sem)
    dcp = pltpu.async_copy(degp_hbm, dg_v, sem)
    ngroups = _stage_edges(ei_hbm, 0, sidx_v, wid)
    _stage_edges(ei_hbm, 1, didx_v, wid)
    _zero_acc_slice(zero_v, acc_sh, sid)
    tcp.wait()
    dcp.wait()
    @plsc.parallel_loop(0, NP // L, 1, unroll=8)
    def _build(i):
        o = i * L
        d = dg_v[0, pl.ds(o, L)] + dg_v[1, pl.ds(o, L)] + 1.0
        t_v[pl.ds(o, L)] = t_v[pl.ds(o, L)] * _fast_rsqrt(d)
    nvec = ngroups * (G * BB // L)

    @plsc.parallel_loop(0, nvec, 1, unroll=8)
    def _gather(i):
        o = i * L
        si = sidx_v[pl.ds(o, L)]
        vals_v[pl.ds(o, L)] = plsc.load_gather(t_v, [si])

    plsc.subcore_barrier()

    def group(g, carry):
        cps = [
            pltpu.async_copy(vals_v.at[pl.ds((g * G + jj) * BB, BB)],
                             acc_sh.at[didx_v.at[pl.ds((g * G + jj) * BB, BB)]],
                             sem, add=True)
            for jj in range(G)
        ]
        for cp in cps:
            cp.wait()
        return carry

    lax.fori_loop(0, ngroups, group, 0)
    plsc.subcore_barrier()
    pltpu.sync_copy(acc_sh.at[pl.ds(sid * NSL, NSL)],
                    out_hbm.at[cid, pl.ds(sid * NSL, NSL)])


def _message_partials(edge_index, t_raw, degp):
    return pl.kernel(
        _msg_body,
        out_type=jax.ShapeDtypeStruct((NC, NP), jnp.float32),
        mesh=_sc_mesh(),
        compiler_params=_SC_PARAMS,
        scratch_types=[
            pltpu.VMEM((EPT,), jnp.int32),
            pltpu.VMEM((EPT,), jnp.int32),
            pltpu.VMEM((EPT,), jnp.float32),
            pltpu.VMEM((NP,), jnp.float32),
            pltpu.VMEM((NC, NP), jnp.float32),
            pltpu.VMEM((NSL,), jnp.float32),
            pltpu.VMEM_SHARED((NP,), jnp.float32),
            pltpu.SemaphoreType.DMA,
        ],
    )(edge_index, t_raw, degp)


def _mid_body(x_ref, wp_ref, bp_ref, wg_ref, wo_ref, traw_ref):
    u = jnp.dot(wo_ref[...][0, :], wg_ref[...])
    h = lax.dot_general(x_ref[...], wp_ref[...],
                        (((1,), (1,)), ((), ())),
                        preferred_element_type=jnp.float32)
    h = jnp.maximum(h + bp_ref[...][None, :], 0.0)
    traw_ref[...] = jnp.sum(h * u[None, :], axis=1)


def _tc_mid(x, W_proj, b_proj, W_gcn, W_out):
    return pl.pallas_call(
        _mid_body,
        grid=(NP // MROWS,),
        in_specs=[
            pl.BlockSpec((MROWS, 128), lambda i: (i, 0)),
            pl.BlockSpec((64, 128), lambda i: (0, 0)),
            pl.BlockSpec((64,), lambda i: (0,)),
            pl.BlockSpec((32, 64), lambda i: (0, 0)),
            pl.BlockSpec((1, 32), lambda i: (0, 0)),
        ],
        out_specs=pl.BlockSpec((MROWS,), lambda i: (i,)),
        out_shape=jax.ShapeDtypeStruct((NP,), jnp.float32),
    )(x, W_proj, b_proj, W_gcn, W_out)


def _final_body(traw_ref, degp_ref, sp_ref, wo_ref, bg_ref, bo_ref, out_ref):
    c = jnp.sum(wo_ref[...][0, :] * bg_ref[...]) + jnp.sum(bo_ref[...])
    deg = degp_ref[0, :] + degp_ref[1, :] + 1.0
    dinv = lax.rsqrt(deg)
    t = dinv * traw_ref[...]
    out_ref[...] = dinv * (sp_ref[0, :] + sp_ref[1, :] + t) + c


def _tc_final(t_raw, degp, sp, W_out, b_gcn, b_out):
    return pl.pallas_call(
        _final_body,
        out_shape=jax.ShapeDtypeStruct((NP,), jnp.float32),
    )(t_raw, degp, sp, W_out, b_gcn, b_out)


@jax.jit
def kernel(x, edge_index, W_proj, b_proj, W_gcn, b_gcn, W_out, b_out):
    degp = _degree_partials(edge_index)
    t_raw = _tc_mid(x, W_proj, b_proj, W_gcn, W_out)
    sp = _message_partials(edge_index, t_raw, degp)
    out_full = _tc_final(t_raw, degp, sp, W_out, b_gcn, b_out)
    return out_full[:N, None]
